# Initial kernel scaffold; baseline (speedup 1.0000x reference)
import jax
import jax.numpy as jnp
from jax.experimental import pallas as pl


def kernel(inputs, edge_index, W1_self, W1_neigh, b1, W2_self, W2_neigh, b2):
    raise NotImplementedError("write your pallas kernel here")


# speedup vs baseline: 3.1487x; 3.1487x over previous
import functools

import jax
import jax.numpy as jnp
from jax import lax
from jax.experimental import pallas as pl
from jax.experimental.pallas import tpu as pltpu
from jax.experimental.pallas import tpu_sc as plsc

_NC = 2
_NS = 16
_NW = _NC * _NS
_L = 16
_CH = 128
_D = 128
_DH = _D // 2


def _sc_segment_sum(src_p, dst_p, table_a, table_b, *, with_deg):
    epad = src_p.shape[0]
    npad = table_a.shape[0]
    ept = epad // _NW
    nchunks = ept // _CH
    rows_per_tile = npad // _NS
    nrowchunks = rows_per_tile // _CH

    out_type = [jax.ShapeDtypeStruct((_NC, npad, _DH), jnp.float32),
                jax.ShapeDtypeStruct((_NC, npad, _DH), jnp.float32)]
    scratch = [
        pltpu.VMEM((_CH,), jnp.int32),
        pltpu.VMEM((_CH,), jnp.int32),
        pltpu.VMEM((_CH, _DH), jnp.float32),
        pltpu.VMEM((_CH, _DH), jnp.float32),
        pltpu.VMEM_SHARED((npad, _DH), jnp.float32),
        pltpu.SemaphoreType.DMA,
    ]
    if with_deg:
        out_type.append(jax.ShapeDtypeStruct((_NC, npad, _L), jnp.float32))
        scratch += [
            pltpu.VMEM((_CH, _L), jnp.float32),
            pltpu.VMEM((rows_per_tile, _L), jnp.float32),
            pltpu.VMEM_SHARED((npad, _L), jnp.float32),
        ]

    mesh = plsc.VectorSubcoreMesh(core_axis_name="c", subcore_axis_name="s")

    @functools.partial(
        pl.kernel, mesh=mesh, out_type=out_type, scratch_types=scratch,
        compiler_params=pltpu.CompilerParams(use_tc_tiling_on_sc=False))
    def body(src_hbm, dst_hbm, ta_hbm, tb_hbm, *refs):
        if with_deg:
            (sa_out, sb_out, deg_out, idxs_v, idxd_v, rows_v, zrows_v,
             acc_sh, sem, ones_v, degstage_v, dega_sh) = refs
        else:
            (sa_out, sb_out, idxs_v, idxd_v, rows_v, zrows_v,
             acc_sh, sem) = refs
        cid = lax.axis_index("c")
        sid = lax.axis_index("s")
        wid = cid * _NS + sid
        zv = jnp.zeros((_L,), jnp.float32)
        row0 = sid * rows_per_tile

        def zrow(k, c):
            zrows_v[k // (_DH // _L), pl.ds((k % (_DH // _L)) * _L, _L)] = zv
            return c
        lax.fori_loop(0, _CH * (_DH // _L), zrow, 0)
        if with_deg:
            ov = jnp.ones((_L,), jnp.float32)

            def fill1(k, c):
                ones_v[k] = ov
                return c
            lax.fori_loop(0, _CH, fill1, 0)

            def zdeg(k, c):
                degstage_v[k] = zv
                return c
            lax.fori_loop(0, rows_per_tile, zdeg, 0)
            pltpu.sync_copy(degstage_v, dega_sh.at[pl.ds(row0, rows_per_tile)])

        base = wid * ept
        for half, (t_hbm, s_out) in enumerate(((ta_hbm, sa_out),
                                               (tb_hbm, sb_out))):
            for k in range(nrowchunks):
                pltpu.sync_copy(zrows_v, acc_sh.at[pl.ds(row0 + k * _CH, _CH)])
            plsc.subcore_barrier()

            deg_now = with_deg and half == 0

            def chunk(i, c):
                off = base + i * _CH
                pltpu.sync_copy(src_hbm.at[pl.ds(off, _CH)], idxs_v)
                pltpu.sync_copy(dst_hbm.at[pl.ds(off, _CH)], idxd_v)
                pltpu.async_copy(t_hbm.at[idxs_v], rows_v, sem).wait()
                pltpu.sync_copy(rows_v, acc_sh.at[idxd_v], add=True)
                if deg_now:
                    pltpu.sync_copy(ones_v, dega_sh.at[idxd_v], add=True)
                return c
            lax.fori_loop(0, nchunks, chunk, 0)
            plsc.subcore_barrier()

            for k in range(nrowchunks):
                r0 = row0 + k * _CH
                pltpu.sync_copy(acc_sh.at[pl.ds(r0, _CH)], rows_v)
                pltpu.sync_copy(rows_v, s_out.at[cid, pl.ds(r0, _CH)])
            if deg_now:
                pltpu.sync_copy(dega_sh.at[pl.ds(row0, rows_per_tile)],
                                degstage_v)
                pltpu.sync_copy(
                    degstage_v,
                    deg_out.at[cid, pl.ds(row0, rows_per_tile)])
            if half == 0:
                plsc.subcore_barrier()

    return body(src_p, dst_p, table_a, table_b)


def _combine_body(h_ref, sa0_ref, sa1_ref, sb0_ref, sb1_ref, da_ref, db_ref,
                  ws_ref, wna_ref, wnb_ref, b_ref, o_ref, *, relu):
    deg = da_ref[...][:, :1] + db_ref[...][:, :1]
    inv = 1.0 / jnp.maximum(deg, 1.0)
    mean_a = (sa0_ref[...] + sa1_ref[...]) * inv
    mean_b = (sb0_ref[...] + sb1_ref[...]) * inv
    acc = jnp.dot(h_ref[...], ws_ref[...], preferred_element_type=jnp.float32)
    acc = acc + jnp.dot(mean_a, wna_ref[...],
                        preferred_element_type=jnp.float32)
    acc = acc + jnp.dot(mean_b, wnb_ref[...],
                        preferred_element_type=jnp.float32)
    acc = acc + b_ref[0:1, :]
    o_ref[...] = jnp.maximum(acc, 0.0) if relu else acc


def _combine(h, s_a, s_b, d_parts, w_self, w_neigh, b, *, relu, bm=1024):
    npad = h.shape[0]
    grid = npad // bm
    b_t = jnp.tile(b[None, :], (8, 1))
    blk = lambda r, c: pl.BlockSpec((r, c), lambda i: (i, 0))
    fix = lambda r, c: pl.BlockSpec((r, c), lambda i: (0, 0))
    return pl.pallas_call(
        functools.partial(_combine_body, relu=relu),
        grid=(grid,),
        in_specs=[
            blk(bm, _D),
            blk(bm, _DH), blk(bm, _DH), blk(bm, _DH), blk(bm, _DH),
            blk(bm, _L), blk(bm, _L),
            fix(_D, _D), fix(_DH, _D), fix(_DH, _D), fix(8, _D),
        ],
        out_specs=blk(bm, _D),
        out_shape=jax.ShapeDtypeStruct((npad, _D), jnp.float32),
    )(h, s_a[0], s_a[1], s_b[0], s_b[1], d_parts[0], d_parts[1],
      w_self, w_neigh[:_DH], w_neigh[_DH:], b_t)


def kernel(inputs, edge_index, W1_self, W1_neigh, b1, W2_self, W2_neigh, b2):
    n, d = inputs.shape
    assert d == _D
    e = edge_index.shape[1]
    npad = -(-n // (_NS * _CH)) * (_NS * _CH)
    epad = -(-e // (_NW * _CH)) * (_NW * _CH)
    src_p = jnp.concatenate(
        [edge_index[0], jnp.zeros((epad - e,), jnp.int32)])
    dst_p = jnp.concatenate(
        [edge_index[1], jnp.full((epad - e,), n, jnp.int32)])
    x = jnp.pad(inputs, ((0, npad - n), (0, 0)))

    s1a, s1b, d1 = _sc_segment_sum(src_p, dst_p, x[:, :_DH], x[:, _DH:],
                                   with_deg=True)
    h1 = _combine(x, s1a, s1b, d1, W1_self, W1_neigh, b1, relu=True)
    s2a, s2b = _sc_segment_sum(src_p, dst_p, h1[:, :_DH], h1[:, _DH:],
                               with_deg=False)
    out = _combine(h1, s2a, s2b, d1, W2_self, W2_neigh, b2, relu=False)
    return out[:n]
